# baseline (speedup 1.0000x reference)
import jax
import jax.numpy as jnp
from jax import lax
from jax.experimental import pallas as pl
from jax.experimental.pallas import tpu as pltpu
from jax.experimental.pallas import tpu_sc as plsc

_NUM_CORES = 2
_NUM_SUBCORES = 16
_NW = _NUM_CORES * _NUM_SUBCORES
_B, _V = 4096, 102
_D = 64
_BPW = _B // _NW
_L = 16
_NJB = _BPW // _L
_NQ = _V // 2


def _sc_body(idx_hbm, table_hbm, out_hbm, idx_v, table_v, bufs, ssems):
    wid = lax.axis_index("s") * _NUM_CORES + lax.axis_index("c")
    bcol = wid * _BPW

    pltpu.sync_copy(idx_hbm.at[:, pl.ds(bcol, _BPW)], idx_v)
    pltpu.sync_copy(table_hbm, table_v)

    def compute_pair(q, buf):
        for k in (0, 1):
            v = q * 2 + k
            for jb in range(_NJB):
                iv = idx_v[v, pl.ds(jb * _L, _L)]

                @plsc.parallel_loop(0, _D, step=1, unroll=16)
                def dloop(d):
                    col = plsc.load_gather(table_v, [iv + d * _V])
                    buf[k, d, pl.ds(jb * _L, _L)] = col

    def out_slice(q):
        return out_hbm.at[pl.ds(q * 2, 2), :, pl.ds(bcol, _BPW)]

    def store(q, b):
        pltpu.async_copy(bufs[b], out_slice(q), ssems[b])

    def wait_store(q, b):
        pltpu.make_async_copy(bufs[b], out_slice(q), ssems[b]).wait()

    for b in (0, 1):
        compute_pair(jnp.int32(b), bufs[b])
        store(b, b)

    def body(r, carry):
        for b in (0, 1):
            q = r * 2 + b
            wait_store(q - 2, b)
            compute_pair(q, bufs[b])
            store(q, b)
        return carry

    lax.fori_loop(1, (_NQ - 1) // 2, body, 0)

    q = _NQ - 1
    wait_store(q - 2, 0)
    compute_pair(q, bufs[0])
    store(q, 0)
    wait_store(_NQ - 2, 1)
    wait_store(_NQ - 1, 0)


@jax.jit
def _lookup(indices, table_flat):
    mesh = plsc.VectorSubcoreMesh(core_axis_name="c", subcore_axis_name="s")
    f = pl.kernel(
        _sc_body,
        out_type=jax.ShapeDtypeStruct((_V, _D, _B), jnp.float32),
        mesh=mesh,
        scratch_types=[
            pltpu.VMEM((_V, _BPW), jnp.int32),
            pltpu.VMEM((_V * _D,), jnp.float32),
            [pltpu.VMEM((2, _D, _BPW), jnp.float32) for _ in range(2)],
            [pltpu.SemaphoreType.DMA for _ in range(2)],
        ],
        compiler_params=pltpu.CompilerParams(
            use_tc_tiling_on_sc=True, needs_layout_passes=False
        ),
    )
    return f(indices, table_flat)


def kernel(indices, table):
    out_t = _lookup(indices.T, table.T.reshape(_V * _D))
    return out_t.transpose(2, 0, 1)


# speedup vs baseline: 1.3885x; 1.3885x over previous
import jax
import jax.numpy as jnp
from jax import lax
from jax.experimental import pallas as pl
from jax.experimental.pallas import tpu as pltpu
from jax.experimental.pallas import tpu_sc as plsc

_NUM_CORES = 2
_NUM_SUBCORES = 16
_NW = _NUM_CORES * _NUM_SUBCORES
_B, _V = 4096, 102
_D = 64
_BPW = _B // _NW
_L = 16
_NJB = _BPW // _L


def _sc_body(idx_hbm, table_hbm, out_hbm, idx_v, table_v, bufs, ssems):
    wid = lax.axis_index("s") * _NUM_CORES + lax.axis_index("c")
    bcol = wid * _BPW

    pltpu.sync_copy(idx_hbm.at[:, pl.ds(bcol, _BPW)], idx_v)
    pltpu.sync_copy(table_hbm, table_v)

    iota = lax.iota(jnp.int32, _L)

    def compute_block(v, buf):
        for jb in range(_NJB):
            lanes = jb * _L + iota
            iv = idx_v[v, pl.ds(jb * _L, _L)]

            @plsc.parallel_loop(0, _D, step=1, unroll=16)
            def dloop(d):
                col = plsc.load_gather(table_v, [iv + d * _V])
                buf[d, pl.ds(jb * _L, _L)] = col

    def out_slice(v):
        return out_hbm.at[v, :, pl.ds(bcol, _BPW)]

    def store(v, b):
        pltpu.async_copy(bufs[b], out_slice(v), ssems[b])

    def wait_store(v, b):
        pltpu.make_async_copy(bufs[b], out_slice(v), ssems[b]).wait()

    for b in (0, 1):
        compute_block(jnp.int32(b), bufs[b])
        store(b, b)

    def body(p, carry):
        for b in (0, 1):
            v = p * 2 + b
            wait_store(v - 2, b)
            compute_block(v, bufs[b])
            store(v, b)
        return carry

    lax.fori_loop(1, _V // 2, body, 0)

    wait_store(_V - 2, 0)
    wait_store(_V - 1, 1)


@jax.jit
def _lookup(indices, table_flat):
    mesh = plsc.VectorSubcoreMesh(core_axis_name="c", subcore_axis_name="s")
    f = pl.kernel(
        _sc_body,
        out_type=jax.ShapeDtypeStruct((_V, _D, _B), jnp.float32),
        mesh=mesh,
        scratch_types=[
            pltpu.VMEM((_V, _BPW), jnp.int32),
            pltpu.VMEM((_V * _D,), jnp.float32),
            [pltpu.VMEM((_D, _BPW), jnp.float32) for _ in range(2)],
            [pltpu.SemaphoreType.DMA for _ in range(2)],
        ],
        compiler_params=pltpu.CompilerParams(
            use_tc_tiling_on_sc=True, needs_layout_passes=False
        ),
    )
    return f(indices, table_flat)


def kernel(indices, table):
    out_t = _lookup(indices.T, table.T.reshape(_V * _D))
    return out_t.transpose(2, 0, 1)
